# baseline (speedup 1.0000x reference)
import functools

import jax
import jax.numpy as jnp
from jax import lax
from jax.experimental import pallas as pl
from jax.experimental.pallas import tpu as pltpu
from jax.experimental.pallas import tpu_sc as plsc

_EPS = 1e-5
_NC = 2
_NS = 16
_NW = _NC * _NS


def _sc_gather_t(emb2_rows, emb1_rows, idx_t):
    R2, V = emb2_rows.shape
    F, B = idx_t.shape
    D = R2 // F
    NJ = R2 // _NW
    GRP = 8
    VC = ((V // 3 + 127) // 128) * 128
    CB = (0, VC, 2 * VC)
    CL = (VC, VC, V - 2 * VC)
    HB = B // 2
    NG = HB // (GRP * 16)

    @functools.partial(
        pl.kernel,
        mesh=plsc.VectorSubcoreMesh(core_axis_name="c", subcore_axis_name="s"),
        compiler_params=pltpu.CompilerParams(use_tc_tiling_on_sc=True,
                                             needs_layout_passes=False),
        out_type=[
            jax.ShapeDtypeStruct((R2, B), jnp.float32),
            jax.ShapeDtypeStruct((F, B), jnp.float32),
        ],
        scratch_types=[
            pltpu.VMEM((CL[0],), jnp.float32),
            pltpu.VMEM((CL[1],), jnp.float32),
            pltpu.VMEM((CL[2],), jnp.float32),
            pltpu.VMEM((B,), jnp.int32),
            pltpu.VMEM((HB,), jnp.float32),
            pltpu.SemaphoreType.DMA,
            pltpu.SemaphoreType.DMA,
            pltpu.SemaphoreType.DMA,
            pltpu.SemaphoreType.DMA,
        ],
    )
    def k(e2_h, e1_h, idx_h, out2_h, out1_h,
          rba, rbb, rbc, idx_v, ob, sema, semb, semc, semw):
        wid = lax.axis_index("s") * _NC + lax.axis_index("c")
        r0 = wid * NJ
        bufs = (rba, rbb, rbc)
        sems = (sema, semb, semc)

        def stream(src_h, r, c):
            return pltpu.async_copy(
                src_h.at[r, pl.ds(CB[c], CL[c])], bufs[c], sems[c])

        def wait_chunk(r, c):
            pltpu.make_async_copy(
                e2_h.at[r, pl.ds(CB[c], CL[c])], bufs[c], sems[c]).wait()

        def wb(r, h):
            return pltpu.async_copy(
                ob, out2_h.at[r, pl.ds(h * HB, HB)], semw)

        def wait_wb(r, h):
            pltpu.make_async_copy(
                ob, out2_h.at[r, pl.ds(h * HB, HB)], semw).wait()

        def pass_c(c, h):
            buf = bufs[c]
            base, ln = CB[c], CL[c]

            def body(i, _):
                off = h * HB + i * (GRP * 16)
                obase = i * (GRP * 16)
                for g in range(GRP):
                    iv = idx_v[pl.ds(off + g * 16, 16)]
                    o = obase + g * 16
                    if c == 0:
                        ivc = jnp.minimum(iv, ln - 1)
                        ob[pl.ds(o, 16)] = plsc.load_gather(buf, [ivc])
                    else:
                        loc = iv - base
                        locc = jnp.maximum(loc, 0)
                        if c == 1:
                            locc = jnp.minimum(locc, ln - 1)
                            keep = (loc >= 0) & (loc < ln)
                        else:
                            keep = loc >= 0
                        v = plsc.load_gather(buf, [locc])
                        cur = ob[pl.ds(o, 16)]
                        ob[pl.ds(o, 16)] = jnp.where(keep, v, cur)
                return 0
            lax.fori_loop(0, NG, body, 0)

        def load_idx(f):
            pltpu.sync_copy(idx_h.at[f], idx_v)

        for c in range(3):
            stream(e2_h, r0, c)

        def row_body(j, carry):
            r = r0 + j
            f = r // D

            @pl.when((j == 0) | (f != (r - 1) // D))
            def _():
                load_idx(f)

            wait_chunk(r, 0)

            @pl.when(j > 0)
            def _():
                wait_wb(r - 1, 1)
            pass_c(0, 0)
            wait_chunk(r, 1)
            pass_c(1, 0)
            wait_chunk(r, 2)
            pass_c(2, 0)
            wb(r, 0)
            wait_wb(r, 0)
            pass_c(0, 1)

            @pl.when(j + 1 < NJ)
            def _():
                stream(e2_h, r + 1, 0)
            pass_c(1, 1)

            @pl.when(j + 1 < NJ)
            def _():
                stream(e2_h, r + 1, 1)
            pass_c(2, 1)

            @pl.when(j + 1 < NJ)
            def _():
                stream(e2_h, r + 1, 2)
            wb(r, 1)
            return carry

        lax.fori_loop(0, NJ, row_body, 0)
        wait_wb(r0 + NJ - 1, 1)

        @pl.when(wid < F)
        def _():
            load_idx(wid)
            for c in range(3):
                pltpu.sync_copy(e1_h.at[wid, pl.ds(CB[c], CL[c])], bufs[c])
            for h in range(2):
                for c in range(3):
                    pass_c(c, h)
                pltpu.sync_copy(ob, out1_h.at[wid, pl.ds(h * HB, HB)])

    return k(emb2_rows, emb1_rows, idx_t)


def _mlp_fused_t(emb_t, fo_t, dense_t, S_t, W0e, W0d, b0c, Wdr, cba,
                 g0c, be0c, W1b, b1c, g1c, be1c, W2b, b2c, g2c, be2c,
                 Wor, cbo, bb):
    E, Bn = emb_t.shape
    H = W0e.shape[0]
    nb = Bn // bb
    inv_b = 1.0 / Bn
    f32 = jnp.float32
    bf16 = jnp.bfloat16

    def body(emb_r, fo_r, dn_r, S_r, W0e_r, W0d_r, b0_r, Wd_r, cba_r,
             g0_r, be0_r, W1_r, b1_r, g1_r, be1_r, W2_r, b2_r, g2_r, be2_r,
             Wo_r, cbo_r, out_r, y0s, y1s, y2s, sts, fof):
        s = pl.program_id(0)
        i = pl.program_id(1)
        sl = pl.ds(i * bb, bb)

        def acc_stats(col, y):
            s1 = jnp.sum(y, axis=1, keepdims=True)
            s2 = jnp.sum(y * y, axis=1, keepdims=True)

            @pl.when(i == 0)
            def _():
                sts[:, col:col + 1] = s1
                sts[:, col + 1:col + 2] = s2

            @pl.when(i != 0)
            def _():
                sts[:, col:col + 1] = sts[:, col:col + 1] + s1
                sts[:, col + 1:col + 2] = sts[:, col + 1:col + 2] + s2

        def bn_relu(ys, col, g_r, be_r):
            mean = sts[:, col:col + 1] * inv_b
            var = sts[:, col + 1:col + 2] * inv_b - mean * mean
            scale = g_r[...] * lax.rsqrt(var + _EPS)
            shift = be_r[...] - mean * scale
            yp = ys[:, sl].astype(f32)
            return jnp.maximum(yp * scale + shift, 0.0)

        @pl.when(s == 0)
        def _():
            x = emb_r[...]
            dn = dn_r[...]
            y = jnp.dot(W0e_r[...], x.astype(bf16), preferred_element_type=f32)
            y = y + jnp.dot(W0d_r[...], dn.astype(bf16),
                            preferred_element_type=f32)
            y = y + b0_r[...]
            y0s[:, sl] = y.astype(bf16)
            acc_stats(0, y)
            se = jnp.dot(S_r[...], x, preferred_element_type=f32)
            sq = jnp.dot(S_r[...], x * x, preferred_element_type=f32)
            fm = 0.5 * jnp.sum(se * se - sq, axis=0, keepdims=True)
            fo = jnp.sum(fo_r[...], axis=0, keepdims=True)
            fo = fo + jnp.dot(Wd_r[...], dn, preferred_element_type=f32)
            fof[:, sl] = fm + fo + cba_r[...]

        @pl.when(s == 1)
        def _():
            h = bn_relu(y0s, 0, g0_r, be0_r)
            y = jnp.dot(W1_r[...], h.astype(bf16),
                        preferred_element_type=f32) + b1_r[...]
            y1s[:, sl] = y.astype(bf16)
            acc_stats(2, y)

        @pl.when(s == 2)
        def _():
            h = bn_relu(y1s, 2, g1_r, be1_r)
            y = jnp.dot(W2_r[...], h.astype(bf16),
                        preferred_element_type=f32) + b2_r[...]
            y2s[:, sl] = y.astype(bf16)
            acc_stats(4, y)

        @pl.when(s == 3)
        def _():
            h = bn_relu(y2s, 4, g2_r, be2_r)
            z = jnp.dot(Wo_r[...], h, preferred_element_type=f32)
            z = z + fof[:, sl] + cbo_r[...]
            out_r[...] = 1.0 / (1.0 + jnp.exp(-z))

    const = lambda shape: pl.BlockSpec(shape, lambda s, i: (0, 0))
    blk = lambda rows: pl.BlockSpec(
        (rows, bb), lambda s, i: (0, jnp.where(s == 0, i, nb - 1)))
    return pl.pallas_call(
        body,
        grid=(4, nb),
        in_specs=[
            blk(E),
            blk(fo_t.shape[0]),
            blk(dense_t.shape[0]),
            const(S_t.shape),
            const(W0e.shape),
            const(W0d.shape),
            const(b0c.shape),
            const(Wdr.shape),
            const(cba.shape),
            const(g0c.shape),
            const(be0c.shape),
            const(W1b.shape),
            const(b1c.shape),
            const(g1c.shape),
            const(be1c.shape),
            const(W2b.shape),
            const(b2c.shape),
            const(g2c.shape),
            const(be2c.shape),
            const(Wor.shape),
            const(cbo.shape),
        ],
        out_specs=pl.BlockSpec((1, bb),
                               lambda s, i: (0, jnp.where(s == 3, i, 0))),
        out_shape=jax.ShapeDtypeStruct((1, Bn), jnp.float32),
        scratch_shapes=[
            pltpu.VMEM((H, Bn), jnp.bfloat16),
            pltpu.VMEM((H, Bn), jnp.bfloat16),
            pltpu.VMEM((H, Bn), jnp.bfloat16),
            pltpu.VMEM((H, 8), jnp.float32),
            pltpu.VMEM((1, Bn), jnp.float32),
        ],
    )(emb_t, fo_t, dense_t, S_t, W0e, W0d, b0c, Wdr, cba,
      g0c, be0c, W1b, b1c, g1c, be1c, W2b, b2c, g2c, be2c, Wor, cbo)


def kernel(sparse_inputs, dense_inputs, emb1, emb2, Wd, bd,
           W0, b0, g0, beta0, W1, b1, g1, beta1, W2, b2, g2, beta2,
           Wo, bo, bias0):
    Bn, F = sparse_inputs.shape
    _, V, D = emb2.shape
    E = F * D

    emb2_rows = emb2.transpose(0, 2, 1).reshape(E, V)
    emb1_rows = emb1.transpose(0, 2, 1).reshape(F, V)
    idx_t = sparse_inputs.T
    dense_t = dense_inputs.T

    emb_t, fo_t = _sc_gather_t(emb2_rows, emb1_rows, idx_t)

    S_t = jnp.tile(jnp.eye(D, dtype=jnp.float32), (1, F))
    W0e = W0[:, :E].astype(jnp.bfloat16)
    W0d = W0[:, E:].astype(jnp.bfloat16)
    W1b = W1.astype(jnp.bfloat16)
    W2b = W2.astype(jnp.bfloat16)
    cbias_a = (bd + bias0).reshape(1, 1)

    bb = 2048
    out = _mlp_fused_t(emb_t, fo_t, dense_t, S_t, W0e, W0d,
                       b0.reshape(-1, 1), Wd, cbias_a,
                       g0.reshape(-1, 1), beta0.reshape(-1, 1),
                       W1b, b1.reshape(-1, 1),
                       g1.reshape(-1, 1), beta1.reshape(-1, 1),
                       W2b, b2.reshape(-1, 1),
                       g2.reshape(-1, 1), beta2.reshape(-1, 1),
                       Wo, bo.reshape(1, 1), bb)
    return out.reshape(Bn)


# speedup vs baseline: 1.4703x; 1.4703x over previous
import functools

import jax
import jax.numpy as jnp
from jax import lax
from jax.experimental import pallas as pl
from jax.experimental.pallas import tpu as pltpu
from jax.experimental.pallas import tpu_sc as plsc

_EPS = 1e-5
_NC = 2
_NS = 16
_NW = _NC * _NS


def _sc_gather_t(emb2_rows, emb1_rows, idx_t):
    R2, V = emb2_rows.shape
    F, B = idx_t.shape
    D = R2 // F
    NJ = R2 // _NW
    QC = 4096
    NQ = B // QC
    GRP = 16

    @functools.partial(
        pl.kernel,
        mesh=plsc.VectorSubcoreMesh(core_axis_name="c", subcore_axis_name="s"),
        compiler_params=pltpu.CompilerParams(use_tc_tiling_on_sc=True,
                                             needs_layout_passes=False),
        out_type=[
            jax.ShapeDtypeStruct((R2, B), jnp.float32),
            jax.ShapeDtypeStruct((F, B), jnp.float32),
        ],
        scratch_types=[
            pltpu.VMEM((V,), jnp.float32),
            pltpu.VMEM((B,), jnp.int32),
            pltpu.VMEM((QC,), jnp.float32),
            pltpu.VMEM((QC,), jnp.float32),
            pltpu.SemaphoreType.DMA,
            pltpu.SemaphoreType.DMA,
            pltpu.SemaphoreType.DMA,
        ],
    )
    def k(e2_h, e1_h, idx_h, out2_h, out1_h,
          row_v, idx_v, ob0, ob1, sem_in, semw0, semw1):
        wid = lax.axis_index("s") * _NC + lax.axis_index("c")
        r0 = wid * NJ
        obufs = (ob0, ob1)
        semws = (semw0, semw1)
        pending = [None, None]

        def gather_quarter(q, ob):
            def body(i, _):
                base = q * QC + i * (GRP * 16)
                obase = i * (GRP * 16)
                for g in range(GRP):
                    iv = idx_v[pl.ds(base + g * 16, 16)]
                    ob[pl.ds(obase + g * 16, 16)] = plsc.load_gather(row_v, [iv])
                return 0
            lax.fori_loop(0, QC // (GRP * 16), body, 0)

        def do_row(out_h, r):
            for q in range(NQ):
                kq = q % 2
                if pending[kq] is not None:
                    pending[kq].wait()
                gather_quarter(q, obufs[kq])
                pending[kq] = pltpu.async_copy(
                    obufs[kq], out_h.at[r, pl.ds(q * QC, QC)], semws[kq])

        for j in range(NJ):
            r = r0 + j
            f = r // D
            if j == 0:
                pltpu.sync_copy(idx_h.at[f], idx_v)
            else:
                fprev = (r0 + j - 1) // D

                @pl.when(f != fprev)
                def _():
                    pltpu.sync_copy(idx_h.at[f], idx_v)

            pltpu.sync_copy(e2_h.at[r], row_v)
            do_row(out2_h, r)

        for kq in range(2):
            if pending[kq] is not None:
                pending[kq].wait()

        @pl.when(wid < F)
        def _():
            pltpu.sync_copy(idx_h.at[wid], idx_v)
            pltpu.sync_copy(e1_h.at[wid], row_v)
            for q in range(NQ):
                gather_quarter(q, obufs[q % 2])
                pltpu.sync_copy(obufs[q % 2],
                                out1_h.at[wid, pl.ds(q * QC, QC)])

    return k(emb2_rows, emb1_rows, idx_t)


def _mlp_fused_t(emb_t, fo_t, dense_t, S_t, W0e, W0d, b0c, Wdr, cba,
                 g0c, be0c, W1b, b1c, g1c, be1c, W2b, b2c, g2c, be2c,
                 Wor, cbo, bb):
    E, Bn = emb_t.shape
    H = W0e.shape[0]
    nb = Bn // bb
    inv_b = 1.0 / Bn
    f32 = jnp.float32
    bf16 = jnp.bfloat16

    def body(emb_r, fo_r, dn_r, S_r, W0e_r, W0d_r, b0_r, Wd_r, cba_r,
             g0_r, be0_r, W1_r, b1_r, g1_r, be1_r, W2_r, b2_r, g2_r, be2_r,
             Wo_r, cbo_r, out_r, y0s, y1s, y2s, sts, fof):
        s = pl.program_id(0)
        i = pl.program_id(1)
        sl = pl.ds(i * bb, bb)

        def acc_stats(col, y):
            s1 = jnp.sum(y, axis=1, keepdims=True)
            s2 = jnp.sum(y * y, axis=1, keepdims=True)

            @pl.when(i == 0)
            def _():
                sts[:, col:col + 1] = s1
                sts[:, col + 1:col + 2] = s2

            @pl.when(i != 0)
            def _():
                sts[:, col:col + 1] = sts[:, col:col + 1] + s1
                sts[:, col + 1:col + 2] = sts[:, col + 1:col + 2] + s2

        def bn_relu(ys, col, g_r, be_r):
            mean = sts[:, col:col + 1] * inv_b
            var = sts[:, col + 1:col + 2] * inv_b - mean * mean
            scale = g_r[...] * lax.rsqrt(var + _EPS)
            shift = be_r[...] - mean * scale
            yp = ys[:, sl].astype(f32)
            return jnp.maximum(yp * scale + shift, 0.0)

        @pl.when(s == 0)
        def _():
            x = emb_r[...]
            dn = dn_r[...]
            y = jnp.dot(W0e_r[...], x.astype(bf16), preferred_element_type=f32)
            y = y + jnp.dot(W0d_r[...], dn.astype(bf16),
                            preferred_element_type=f32)
            y = y + b0_r[...]
            y0s[:, sl] = y.astype(bf16)
            acc_stats(0, y)
            se = jnp.dot(S_r[...], x, preferred_element_type=f32)
            sq = jnp.dot(S_r[...], x * x, preferred_element_type=f32)
            fm = 0.5 * jnp.sum(se * se - sq, axis=0, keepdims=True)
            fo = jnp.sum(fo_r[...], axis=0, keepdims=True)
            fo = fo + jnp.dot(Wd_r[...], dn, preferred_element_type=f32)
            fof[:, sl] = fm + fo + cba_r[...]

        @pl.when(s == 1)
        def _():
            h = bn_relu(y0s, 0, g0_r, be0_r)
            y = jnp.dot(W1_r[...], h.astype(bf16),
                        preferred_element_type=f32) + b1_r[...]
            y1s[:, sl] = y.astype(bf16)
            acc_stats(2, y)

        @pl.when(s == 2)
        def _():
            h = bn_relu(y1s, 2, g1_r, be1_r)
            y = jnp.dot(W2_r[...], h.astype(bf16),
                        preferred_element_type=f32) + b2_r[...]
            y2s[:, sl] = y.astype(bf16)
            acc_stats(4, y)

        @pl.when(s == 3)
        def _():
            h = bn_relu(y2s, 4, g2_r, be2_r)
            z = jnp.dot(Wo_r[...], h, preferred_element_type=f32)
            z = z + fof[:, sl] + cbo_r[...]
            out_r[...] = 1.0 / (1.0 + jnp.exp(-z))

    const = lambda shape: pl.BlockSpec(shape, lambda s, i: (0, 0))
    blk = lambda rows: pl.BlockSpec(
        (rows, bb), lambda s, i: (0, jnp.where(s == 0, i, nb - 1)))
    return pl.pallas_call(
        body,
        grid=(4, nb),
        in_specs=[
            blk(E),
            blk(fo_t.shape[0]),
            blk(dense_t.shape[0]),
            const(S_t.shape),
            const(W0e.shape),
            const(W0d.shape),
            const(b0c.shape),
            const(Wdr.shape),
            const(cba.shape),
            const(g0c.shape),
            const(be0c.shape),
            const(W1b.shape),
            const(b1c.shape),
            const(g1c.shape),
            const(be1c.shape),
            const(W2b.shape),
            const(b2c.shape),
            const(g2c.shape),
            const(be2c.shape),
            const(Wor.shape),
            const(cbo.shape),
        ],
        out_specs=pl.BlockSpec((1, bb),
                               lambda s, i: (0, jnp.where(s == 3, i, 0))),
        out_shape=jax.ShapeDtypeStruct((1, Bn), jnp.float32),
        scratch_shapes=[
            pltpu.VMEM((H, Bn), jnp.bfloat16),
            pltpu.VMEM((H, Bn), jnp.bfloat16),
            pltpu.VMEM((H, Bn), jnp.bfloat16),
            pltpu.VMEM((H, 8), jnp.float32),
            pltpu.VMEM((1, Bn), jnp.float32),
        ],
    )(emb_t, fo_t, dense_t, S_t, W0e, W0d, b0c, Wdr, cba,
      g0c, be0c, W1b, b1c, g1c, be1c, W2b, b2c, g2c, be2c, Wor, cbo)


def kernel(sparse_inputs, dense_inputs, emb1, emb2, Wd, bd,
           W0, b0, g0, beta0, W1, b1, g1, beta1, W2, b2, g2, beta2,
           Wo, bo, bias0):
    Bn, F = sparse_inputs.shape
    _, V, D = emb2.shape
    E = F * D

    emb2_rows = emb2.transpose(0, 2, 1).reshape(E, V)
    emb1_rows = emb1.transpose(0, 2, 1).reshape(F, V)
    idx_t = sparse_inputs.T
    dense_t = dense_inputs.T

    emb_t, fo_t = _sc_gather_t(emb2_rows, emb1_rows, idx_t)

    S_t = jnp.tile(jnp.eye(D, dtype=jnp.float32), (1, F))
    W0e = W0[:, :E].astype(jnp.bfloat16)
    W0d = W0[:, E:].astype(jnp.bfloat16)
    W1b = W1.astype(jnp.bfloat16)
    W2b = W2.astype(jnp.bfloat16)
    cbias_a = (bd + bias0).reshape(1, 1)

    bb = 2048
    out = _mlp_fused_t(emb_t, fo_t, dense_t, S_t, W0e, W0d,
                       b0.reshape(-1, 1), Wd, cbias_a,
                       g0.reshape(-1, 1), beta0.reshape(-1, 1),
                       W1b, b1.reshape(-1, 1),
                       g1.reshape(-1, 1), beta1.reshape(-1, 1),
                       W2b, b2.reshape(-1, 1),
                       g2.reshape(-1, 1), beta2.reshape(-1, 1),
                       Wo, bo.reshape(1, 1), bb)
    return out.reshape(Bn)


# speedup vs baseline: 1.4880x; 1.0121x over previous
import functools

import jax
import jax.numpy as jnp
from jax import lax
from jax.experimental import pallas as pl
from jax.experimental.pallas import tpu as pltpu
from jax.experimental.pallas import tpu_sc as plsc

_EPS = 1e-5
_NC = 2
_NS = 16
_NW = _NC * _NS


def _sc_gather_t(emb2_rows, emb1_rows, idx_t):
    R2, V = emb2_rows.shape
    F, B = idx_t.shape
    D = R2 // F
    NJ = R2 // _NW
    QC = 4096
    NQ = B // QC
    GRP = 8

    @functools.partial(
        pl.kernel,
        mesh=plsc.VectorSubcoreMesh(core_axis_name="c", subcore_axis_name="s"),
        compiler_params=pltpu.CompilerParams(use_tc_tiling_on_sc=True,
                                             needs_layout_passes=False),
        out_type=[
            jax.ShapeDtypeStruct((R2, B), jnp.float32),
            jax.ShapeDtypeStruct((F, B), jnp.float32),
        ],
        scratch_types=[
            pltpu.VMEM((V,), jnp.float32),
            pltpu.VMEM((B,), jnp.int32),
            pltpu.VMEM((QC,), jnp.float32),
            pltpu.VMEM((QC,), jnp.float32),
            pltpu.SemaphoreType.DMA,
            pltpu.SemaphoreType.DMA,
            pltpu.SemaphoreType.DMA,
        ],
    )
    def k(e2_h, e1_h, idx_h, out2_h, out1_h,
          row_v, idx_v, ob0, ob1, sem_in, semw0, semw1):
        wid = lax.axis_index("s") * _NC + lax.axis_index("c")
        r0 = wid * NJ
        obufs = (ob0, ob1)
        semws = (semw0, semw1)
        pending = [None, None]

        def gather_quarter(q, ob):
            def body(i, _):
                base = q * QC + i * (GRP * 16)
                obase = i * (GRP * 16)
                for g in range(GRP):
                    iv = idx_v[pl.ds(base + g * 16, 16)]
                    ob[pl.ds(obase + g * 16, 16)] = plsc.load_gather(row_v, [iv])
                return 0
            lax.fori_loop(0, QC // (GRP * 16), body, 0)

        def do_row(out_h, r):
            for q in range(NQ):
                kq = q % 2
                if pending[kq] is not None:
                    pending[kq].wait()
                gather_quarter(q, obufs[kq])
                pending[kq] = pltpu.async_copy(
                    obufs[kq], out_h.at[r, pl.ds(q * QC, QC)], semws[kq])

        for j in range(NJ):
            r = r0 + j
            f = r // D
            if j == 0:
                pltpu.sync_copy(idx_h.at[f], idx_v)
            else:
                fprev = (r0 + j - 1) // D

                @pl.when(f != fprev)
                def _():
                    pltpu.sync_copy(idx_h.at[f], idx_v)

            pltpu.sync_copy(e2_h.at[r], row_v)
            do_row(out2_h, r)

        for kq in range(2):
            if pending[kq] is not None:
                pending[kq].wait()

        @pl.when(wid < F)
        def _():
            pltpu.sync_copy(idx_h.at[wid], idx_v)
            pltpu.sync_copy(e1_h.at[wid], row_v)
            pend = [None, None]
            for q in range(NQ):
                kq = q % 2
                if pend[kq] is not None:
                    pend[kq].wait()
                gather_quarter(q, obufs[kq])
                pend[kq] = pltpu.async_copy(
                    obufs[kq], out1_h.at[wid, pl.ds(q * QC, QC)], semws[kq])
            for kq in range(2):
                if pend[kq] is not None:
                    pend[kq].wait()

    return k(emb2_rows, emb1_rows, idx_t)


def _mlp_fused_t(emb_t, fo_t, dense_t, S_t, W0e, W0d, b0c, Wdr, cba,
                 g0c, be0c, W1b, b1c, g1c, be1c, W2b, b2c, g2c, be2c,
                 Wor, cbo, bb):
    E, Bn = emb_t.shape
    H = W0e.shape[0]
    nb = Bn // bb
    inv_b = 1.0 / Bn
    f32 = jnp.float32
    bf16 = jnp.bfloat16

    def body(emb_r, fo_r, dn_r, S_r, W0e_r, W0d_r, b0_r, Wd_r, cba_r,
             g0_r, be0_r, W1_r, b1_r, g1_r, be1_r, W2_r, b2_r, g2_r, be2_r,
             Wo_r, cbo_r, out_r, y0s, y1s, y2s, sts, fof):
        s = pl.program_id(0)
        i = pl.program_id(1)
        sl = pl.ds(i * bb, bb)

        def acc_stats(col, y):
            s1 = jnp.sum(y, axis=1, keepdims=True)
            s2 = jnp.sum(y * y, axis=1, keepdims=True)

            @pl.when(i == 0)
            def _():
                sts[:, col:col + 1] = s1
                sts[:, col + 1:col + 2] = s2

            @pl.when(i != 0)
            def _():
                sts[:, col:col + 1] = sts[:, col:col + 1] + s1
                sts[:, col + 1:col + 2] = sts[:, col + 1:col + 2] + s2

        def bn_relu(ys, col, g_r, be_r):
            mean = sts[:, col:col + 1] * inv_b
            var = sts[:, col + 1:col + 2] * inv_b - mean * mean
            scale = g_r[...] * lax.rsqrt(var + _EPS)
            shift = be_r[...] - mean * scale
            yp = ys[:, sl].astype(f32)
            return jnp.maximum(yp * scale + shift, 0.0)

        @pl.when(s == 0)
        def _():
            x = emb_r[...]
            dn = dn_r[...]
            y = jnp.dot(W0e_r[...], x.astype(bf16), preferred_element_type=f32)
            y = y + jnp.dot(W0d_r[...], dn.astype(bf16),
                            preferred_element_type=f32)
            y = y + b0_r[...]
            y0s[:, sl] = y.astype(bf16)
            acc_stats(0, y)
            se = jnp.dot(S_r[...], x, preferred_element_type=f32)
            sq = jnp.dot(S_r[...], x * x, preferred_element_type=f32)
            fm = 0.5 * jnp.sum(se * se - sq, axis=0, keepdims=True)
            fo = jnp.sum(fo_r[...], axis=0, keepdims=True)
            fo = fo + jnp.dot(Wd_r[...], dn, preferred_element_type=f32)
            fof[:, sl] = fm + fo + cba_r[...]

        @pl.when(s == 1)
        def _():
            h = bn_relu(y0s, 0, g0_r, be0_r)
            y = jnp.dot(W1_r[...], h.astype(bf16),
                        preferred_element_type=f32) + b1_r[...]
            y1s[:, sl] = y.astype(bf16)
            acc_stats(2, y)

        @pl.when(s == 2)
        def _():
            h = bn_relu(y1s, 2, g1_r, be1_r)
            y = jnp.dot(W2_r[...], h.astype(bf16),
                        preferred_element_type=f32) + b2_r[...]
            y2s[:, sl] = y.astype(bf16)
            acc_stats(4, y)

        @pl.when(s == 3)
        def _():
            h = bn_relu(y2s, 4, g2_r, be2_r)
            z = jnp.dot(Wo_r[...], h, preferred_element_type=f32)
            z = z + fof[:, sl] + cbo_r[...]
            out_r[...] = 1.0 / (1.0 + jnp.exp(-z))

    const = lambda shape: pl.BlockSpec(shape, lambda s, i: (0, 0))
    blk = lambda rows: pl.BlockSpec(
        (rows, bb), lambda s, i: (0, jnp.where(s == 0, i, nb - 1)))
    return pl.pallas_call(
        body,
        grid=(4, nb),
        in_specs=[
            blk(E),
            blk(fo_t.shape[0]),
            blk(dense_t.shape[0]),
            const(S_t.shape),
            const(W0e.shape),
            const(W0d.shape),
            const(b0c.shape),
            const(Wdr.shape),
            const(cba.shape),
            const(g0c.shape),
            const(be0c.shape),
            const(W1b.shape),
            const(b1c.shape),
            const(g1c.shape),
            const(be1c.shape),
            const(W2b.shape),
            const(b2c.shape),
            const(g2c.shape),
            const(be2c.shape),
            const(Wor.shape),
            const(cbo.shape),
        ],
        out_specs=pl.BlockSpec((1, bb),
                               lambda s, i: (0, jnp.where(s == 3, i, 0))),
        out_shape=jax.ShapeDtypeStruct((1, Bn), jnp.float32),
        scratch_shapes=[
            pltpu.VMEM((H, Bn), jnp.bfloat16),
            pltpu.VMEM((H, Bn), jnp.bfloat16),
            pltpu.VMEM((H, Bn), jnp.bfloat16),
            pltpu.VMEM((H, 8), jnp.float32),
            pltpu.VMEM((1, Bn), jnp.float32),
        ],
    )(emb_t, fo_t, dense_t, S_t, W0e, W0d, b0c, Wdr, cba,
      g0c, be0c, W1b, b1c, g1c, be1c, W2b, b2c, g2c, be2c, Wor, cbo)


def kernel(sparse_inputs, dense_inputs, emb1, emb2, Wd, bd,
           W0, b0, g0, beta0, W1, b1, g1, beta1, W2, b2, g2, beta2,
           Wo, bo, bias0):
    Bn, F = sparse_inputs.shape
    _, V, D = emb2.shape
    E = F * D

    emb2_rows = emb2.transpose(0, 2, 1).reshape(E, V)
    emb1_rows = emb1.transpose(0, 2, 1).reshape(F, V)
    idx_t = sparse_inputs.T
    dense_t = dense_inputs.T

    emb_t, fo_t = _sc_gather_t(emb2_rows, emb1_rows, idx_t)

    S_t = jnp.tile(jnp.eye(D, dtype=jnp.float32), (1, F))
    W0e = W0[:, :E].astype(jnp.bfloat16)
    W0d = W0[:, E:].astype(jnp.bfloat16)
    W1b = W1.astype(jnp.bfloat16)
    W2b = W2.astype(jnp.bfloat16)
    cbias_a = (bd + bias0).reshape(1, 1)

    bb = 2048
    out = _mlp_fused_t(emb_t, fo_t, dense_t, S_t, W0e, W0d,
                       b0.reshape(-1, 1), Wd, cbias_a,
                       g0.reshape(-1, 1), beta0.reshape(-1, 1),
                       W1b, b1.reshape(-1, 1),
                       g1.reshape(-1, 1), beta1.reshape(-1, 1),
                       W2b, b2.reshape(-1, 1),
                       g2.reshape(-1, 1), beta2.reshape(-1, 1),
                       Wo, bo.reshape(1, 1), bb)
    return out.reshape(Bn)
